# baseline (speedup 1.0000x reference)
import jax
import jax.numpy as jnp
import numpy as np
from jax import lax
from jax.experimental import pallas as pl
from jax.experimental.pallas import tpu as pltpu

B = 16
C = 32
N = 32
J = 64
CHUNK = 512
K = J * CHUNK

_GUMBEL_CBN = np.asarray(
    jax.random.gumbel(jax.random.key(123), (B, C, N), jnp.float32)
).transpose(1, 0, 2).copy()


def _gather_body(filt_ref, x_ref, xs_ref, s_ref, acc):
    j = pl.program_id(0)

    @pl.when(j == 0)
    def _():
        acc[...] = jnp.zeros_like(acc)

    xb = x_ref[:, 0, 0, :]
    xs_ref[:, 0, 0, :] = xb
    acc[...] += jnp.sum(xb, axis=1, keepdims=True)

    @pl.when(j == J - 1)
    def _():
        s_ref[...] = acc[...]


def _main_body(s_ref, xs_ref, w_ref, g_ref, o_ref, xsn, h3):
    c = pl.program_id(0)

    @pl.when(c == 0)
    def _():
        s = s_ref[...]
        xsn[...] = jnp.where(s > 0.0, xs_ref[...] / s, 0.0)

    h3[c, :, :] = jnp.dot(xsn[...], w_ref[0], preferred_element_type=jnp.float32)

    @pl.when(c == C - 1)
    def _():
        total = jnp.float32(0.0)
        for cc in range(C):
            total += jnp.sum(jnp.max(h3[cc, :, :], axis=1))
        avg = total / jnp.float32(B * C)
        temp = 1.0 / (avg + jnp.float32(0.0001)) - 1.0
        iota2 = lax.broadcasted_iota(jnp.int32, (B, N), 1)
        for cc in range(C):
            z = h3[cc, :, :] / temp + g_ref[cc, :, :]
            m = jnp.max(z, axis=1, keepdims=True)
            cand = jnp.where(z == m, iota2, N)
            am = jnp.min(cand, axis=1, keepdims=True)
            o_ref[:, cc, :] = (iota2 == am).astype(jnp.int32)


def kernel(x, weights, input_filter):
    x4 = x.reshape(B, 1024, 1, CHUNK)
    g3 = jnp.asarray(_GUMBEL_CBN)

    xs4, row_sums = pl.pallas_call(
        _gather_body,
        grid_spec=pltpu.PrefetchScalarGridSpec(
            num_scalar_prefetch=1,
            grid=(J,),
            in_specs=[
                pl.BlockSpec((B, 1, 1, CHUNK),
                             lambda j, filt: (0, filt[j], 0, 0)),
            ],
            out_specs=[
                pl.BlockSpec((B, 1, 1, CHUNK), lambda j, filt: (0, j, 0, 0)),
                pl.BlockSpec((B, 1), lambda j, filt: (0, 0)),
            ],
            scratch_shapes=[pltpu.VMEM((B, 1), jnp.float32)],
        ),
        out_shape=[
            jax.ShapeDtypeStruct((B, J, 1, CHUNK), jnp.float32),
            jax.ShapeDtypeStruct((B, 1), jnp.float32),
        ],
    )(input_filter, x4)

    xs2 = xs4.reshape(B, K)

    out = pl.pallas_call(
        _main_body,
        grid=(C,),
        in_specs=[
            pl.BlockSpec((B, 1), lambda c: (0, 0)),
            pl.BlockSpec((B, K), lambda c: (0, 0)),
            pl.BlockSpec((1, K, N), lambda c: (c, 0, 0)),
            pl.BlockSpec((C, B, N), lambda c: (0, 0, 0)),
        ],
        out_specs=pl.BlockSpec((B, C, N), lambda c: (0, 0, 0)),
        scratch_shapes=[
            pltpu.VMEM((B, K), jnp.float32),
            pltpu.VMEM((C, B, N), jnp.float32),
        ],
        out_shape=jax.ShapeDtypeStruct((B, C, N), jnp.int32),
    )(row_sums, xs2, weights, g3)

    return out
